# baseline (speedup 1.0000x reference)
import functools

import jax
import jax.numpy as jnp
from jax import lax
from jax.experimental import pallas as pl

NC = 32
C = 256
D = 16
K = 16
ALPHA = 8
KB = 8
HS = 32
HM = 32
HMOD = 32
MOD_IN = K + 3 * D + 1
MOD_OUT = K + KB + 1 + D
R0 = C - ALPHA
CH = 128
CL = R0 - CH


def _body(xa_ref, h_ref, conn_ref, gate_ref, prev_ref,
          hebb_ref, decay_ref, prim_ref, nid_ref,
          m1_ref, mb1_ref, m2_ref, mb2_ref,
          sw1_ref, sb1_ref, sw2_ref, sb2_ref,
          mw1_ref, mb1s_ref, mw2_ref, mb2s_ref,
          out_ref, *, bs):
    f32 = jnp.float32
    i = pl.program_id(0)

    BR = bs * ALPHA
    ht = h_ref[...].reshape(bs, D, C)
    gate = jax.nn.sigmoid(gate_ref[...].reshape(bs, K, C)[:, :, R0:])
    idx = conn_ref[...].reshape(K, C)[:, R0:]

    def cols(a):
        return jnp.concatenate([a[b] for b in range(bs)], axis=1)

    def tile8(a):
        return jnp.concatenate([a] * bs, axis=-1)

    xa = xa_ref[...]
    cell_mask = (lax.broadcasted_iota(jnp.int32, (1, 1, 1, NC), 3) == i
                 ).astype(f32)
    x_c = (xa * cell_mask).sum(axis=3)

    idx64 = tile8(idx)
    gate_cols = cols(gate)
    ciota = lax.broadcasted_iota(jnp.int32, (C, BR), 0)
    m_parts = [jnp.zeros((C, BR), f32) for _ in range(4)]
    for k in range(K):
        sel = jnp.where(idx64[k:k + 1, :] == ciota, gate_cols[k:k + 1, :],
                        jnp.zeros((), f32))
        m_parts[k % 4] = m_parts[k % 4] + sel
    m_mix = (m_parts[0] + m_parts[1]) + (m_parts[2] + m_parts[3])
    gath_list = []
    for b in range(bs):
        m_b = m_mix[:, b * ALPHA:(b + 1) * ALPHA]
        g = jnp.dot(ht[b], m_b, preferred_element_type=f32)
        g = g + lax.dot_general(x_c[b], m_b[:ALPHA, :],
                                (((0,), (0,)), ((), ())),
                                preferred_element_type=f32)
        gath_list.append(g)
    gath_cols = jnp.concatenate(gath_list, axis=1)

    ht_cols = cols(ht[:, :, R0:])
    prev_cols = cols(prev_ref[...].reshape(bs, D, C)[:, :, R0:])

    minp = jnp.concatenate([ht_cols, gath_cols, prev_cols], axis=0)
    mh = jnp.tanh(jnp.dot(mw1_ref[...], minp, preferred_element_type=f32)
                  + mb1s_ref[...])
    msgt = (jnp.dot(mw2_ref[...], mh, preferred_element_type=f32)
            + mb2s_ref[...])

    hebb_cols = cols(hebb_ref[...].reshape(bs, K, C)[:, :, R0:])
    prim_cols = cols(prim_ref[...].reshape(bs, D, C)[:, :, R0:])
    dec_cols = cols(decay_ref[...].reshape(bs, 1, C)[:, :, R0:])
    nid_cols = tile8(nid_ref[...].reshape(D, C)[:, R0:])
    inp_mod = jnp.concatenate(
        [hebb_cols, ht_cols, dec_cols, prim_cols, nid_cols], axis=0)

    w1_64 = tile8(m1_ref[...].reshape(MOD_IN, HMOD, CH)[:, :, CL:])
    w2_64 = tile8(m2_ref[...].reshape(MOD_OUT, HMOD, CH)[:, :, CL:])
    h_parts = [tile8(mb1_ref[...][:, CL:])] + \
              [jnp.zeros((HMOD, BR), f32) for _ in range(3)]
    for ii in range(MOD_IN):
        h_parts[ii % 4] = (h_parts[ii % 4]
                           + inp_mod[ii:ii + 1, :] * w1_64[ii])
    hid = jnp.tanh((h_parts[0] + h_parts[1]) + (h_parts[2] + h_parts[3]))
    o_parts = [tile8(mb2_ref[...][:, CL:])] + \
              [jnp.zeros((MOD_OUT, BR), f32) for _ in range(3)]
    for hh in range(HMOD):
        o_parts[hh % 4] = (o_parts[hh % 4]
                           + hid[hh:hh + 1, :] * w2_64[:, hh, :])
    outm = (o_parts[0] + o_parts[1]) + (o_parts[2] + o_parts[3])

    nd = outm[K + KB:K + KB + 1, :]
    new_prim = outm[K + KB + 1:, :]

    sinp = jnp.concatenate([ht_cols, msgt, new_prim, nd], axis=0)
    sh = jnp.tanh(jnp.dot(sw1_ref[...], sinp, preferred_element_type=f32)
                  + sb1_ref[...])
    delta = (jnp.dot(sw2_ref[...], sh, preferred_element_type=f32)
             + sb2_ref[...])

    h_new = ht_cols * jax.nn.sigmoid(nd) + delta
    out_ref[...] = h_new.reshape(1, D, BR)


def kernel(x, h, prev_messages, w_conn, decay_logit, primitives_state,
           hebbian_traces, state_w1, state_b1, state_w2, state_b2,
           msg_w1, msg_b1, msg_w2, msg_b2,
           mod_w1, mod_b1, mod_w2, mod_b2,
           neuron_id, conn_indices, border_indices):
    bs = x.shape[0]
    N = NC * C

    xv = x.transpose(0, 2, 3, 1)
    ht = h.transpose(0, 1, 3, 2)
    prevt = prev_messages.transpose(0, 1, 3, 2)
    wct = w_conn.transpose(0, 1, 3, 2)
    hebbt = hebbian_traces.transpose(0, 1, 3, 2)
    primt = primitives_state.transpose(0, 1, 3, 2)
    nidt = neuron_id.transpose(0, 2, 1)
    connt = conn_indices.transpose(0, 2, 1)
    m1t = mod_w1.transpose(2, 1, 0).reshape(MOD_IN * HMOD, N)
    m2t = mod_w2.transpose(2, 1, 0).reshape(MOD_OUT * HMOD, N)
    mb1t = mod_b1.transpose(1, 0)
    mb2t = mod_b2.transpose(1, 0)
    sb1c = state_b1.reshape(HS, 1)
    sb2c = state_b2.reshape(D, 1)
    mb1c = msg_b1.reshape(HM, 1)
    mb2c = msg_b2.reshape(D, 1)

    grid = (NC,)
    body = functools.partial(_body, bs=bs)
    out = pl.pallas_call(
        body,
        grid=grid,
        in_specs=[
            pl.BlockSpec((bs, ALPHA, D, NC), lambda i: (0, 0, 0, 0)),
            pl.BlockSpec((bs, 1, D, C), lambda i: (0, i, 0, 0)),
            pl.BlockSpec((1, K, C), lambda i: (i, 0, 0)),
            pl.BlockSpec((bs, 1, K, C), lambda i: (0, i, 0, 0)),
            pl.BlockSpec((bs, 1, D, C), lambda i: (0, i, 0, 0)),
            pl.BlockSpec((bs, 1, K, C), lambda i: (0, i, 0, 0)),
            pl.BlockSpec((bs, 1, 1, C), lambda i: (0, i, 0, 0)),
            pl.BlockSpec((bs, 1, D, C), lambda i: (0, i, 0, 0)),
            pl.BlockSpec((1, D, C), lambda i: (i, 0, 0)),
            pl.BlockSpec((MOD_IN * HMOD, CH), lambda i: (0, 2 * i + 1)),
            pl.BlockSpec((HMOD, CH), lambda i: (0, 2 * i + 1)),
            pl.BlockSpec((MOD_OUT * HMOD, CH), lambda i: (0, 2 * i + 1)),
            pl.BlockSpec((MOD_OUT, CH), lambda i: (0, 2 * i + 1)),
            pl.BlockSpec(state_w1.shape, lambda i: (0, 0)),
            pl.BlockSpec((HS, 1), lambda i: (0, 0)),
            pl.BlockSpec(state_w2.shape, lambda i: (0, 0)),
            pl.BlockSpec((D, 1), lambda i: (0, 0)),
            pl.BlockSpec(msg_w1.shape, lambda i: (0, 0)),
            pl.BlockSpec((HM, 1), lambda i: (0, 0)),
            pl.BlockSpec(msg_w2.shape, lambda i: (0, 0)),
            pl.BlockSpec((D, 1), lambda i: (0, 0)),
        ],
        out_specs=pl.BlockSpec((1, D, bs * ALPHA), lambda i: (i, 0, 0)),
        out_shape=jax.ShapeDtypeStruct((NC, D, bs * ALPHA), jnp.float32),
    )(xv, ht, connt, wct, prevt, hebbt,
      decay_logit.reshape(bs, NC, 1, C), primt, nidt,
      m1t, mb1t, m2t, mb2t,
      state_w1, sb1c, state_w2, sb2c,
      msg_w1, mb1c, msg_w2, mb2c)
    return out.reshape(NC, D, bs, ALPHA).transpose(2, 0, 3, 1)


# speedup vs baseline: 1.5571x; 1.5571x over previous
import functools

import jax
import jax.numpy as jnp
from jax import lax
from jax.experimental import pallas as pl

NC = 32
C = 256
D = 16
K = 16
ALPHA = 8
KB = 8
HS = 32
HM = 32
HMOD = 32
MOD_IN = K + 3 * D + 1
MOD_OUT = K + KB + 1 + D
R0 = C - ALPHA
CH = 128
CL = R0 - CH


def _body(xa_ref, h_ref, conn_ref, gate_ref, prev_ref,
          hebb_ref, decay_ref, prim_ref, nid_ref,
          m1_ref, mb1_ref, m2_ref, mb2_ref,
          sw1_ref, sb1_ref, sw2_ref, sb2_ref,
          mw1_ref, mb1s_ref, mw2_ref, mb2s_ref,
          out_ref, *, bs):
    f32 = jnp.float32
    i = pl.program_id(0)

    ht = h_ref[...].reshape(bs, D, C)
    gate = jax.nn.sigmoid(gate_ref[...].reshape(bs, K, C)[:, :, R0:])
    idx = conn_ref[...].reshape(K, C)[:, R0:]

    xa = xa_ref[...]
    cell_mask = (lax.broadcasted_iota(jnp.int32, (1, 1, 1, NC), 3) == i
                 ).astype(f32)
    x_c = (xa * cell_mask).sum(axis=3)

    ciota = lax.broadcasted_iota(jnp.int32, (C, ALPHA), 0)
    m_parts = [jnp.zeros((bs, C, ALPHA), f32) for _ in range(4)]
    for k in range(K):
        oh_k = (idx[k:k + 1, :] == ciota).astype(f32)
        m_parts[k % 4] = m_parts[k % 4] + oh_k[None] * gate[:, k:k + 1, :]
    m_mix = (m_parts[0] + m_parts[1]) + (m_parts[2] + m_parts[3])
    gath_list = []
    for b in range(bs):
        g = jnp.dot(ht[b], m_mix[b], preferred_element_type=f32)
        g = g + lax.dot_general(x_c[b], m_mix[b][:ALPHA, :],
                                (((0,), (0,)), ((), ())),
                                preferred_element_type=f32)
        gath_list.append(g)

    ht_r = ht[:, :, R0:]
    prev_r = prev_ref[...].reshape(bs, D, C)[:, :, R0:]

    inp_cols = [jnp.concatenate([ht_r[b], gath_list[b], prev_r[b]], axis=0)
                for b in range(bs)]
    minp = jnp.concatenate(inp_cols, axis=1)
    mh = jnp.tanh(jnp.dot(mw1_ref[...], minp, preferred_element_type=f32)
                  + mb1s_ref[...])
    msgt = (jnp.dot(mw2_ref[...], mh, preferred_element_type=f32)
            + mb2s_ref[...])

    hebb_r = hebb_ref[...].reshape(bs, K, C)[:, :, R0:]
    prim_r = prim_ref[...].reshape(bs, D, C)[:, :, R0:]
    dec_r = decay_ref[...].reshape(bs, 1, C)[:, :, R0:]
    nid_r = jnp.broadcast_to(
        nid_ref[...].reshape(D, C)[None, :, R0:], (bs, D, ALPHA))
    inp_mod = jnp.concatenate(
        [hebb_r, ht_r, dec_r, prim_r, nid_r], axis=1)

    w1 = m1_ref[...].reshape(MOD_IN, HMOD, CH)
    w2 = m2_ref[...].reshape(MOD_OUT, HMOD, CH)
    h_parts = [jnp.broadcast_to(mb1_ref[...][None, :, CL:],
                                (bs, HMOD, ALPHA))] + \
              [jnp.zeros((bs, HMOD, ALPHA), f32) for _ in range(3)]
    for ii in range(MOD_IN):
        h_parts[ii % 4] = (h_parts[ii % 4]
                           + inp_mod[:, ii:ii + 1, :] * w1[ii][None, :, CL:])
    hid = jnp.tanh((h_parts[0] + h_parts[1]) + (h_parts[2] + h_parts[3]))
    o_parts = [jnp.broadcast_to(mb2_ref[...][None, :, CL:],
                                (bs, MOD_OUT, ALPHA))] + \
              [jnp.zeros((bs, MOD_OUT, ALPHA), f32) for _ in range(3)]
    for hh in range(HMOD):
        o_parts[hh % 4] = (o_parts[hh % 4]
                           + hid[:, hh:hh + 1, :] * w2[:, hh, CL:][None])
    outm = (o_parts[0] + o_parts[1]) + (o_parts[2] + o_parts[3])

    nd = outm[:, K + KB:K + KB + 1, :]
    new_prim = outm[:, K + KB + 1:, :]

    st_cols = [jnp.concatenate(
        [ht_r[b], msgt[:, b * ALPHA:(b + 1) * ALPHA], new_prim[b], nd[b]],
        axis=0) for b in range(bs)]
    sinp = jnp.concatenate(st_cols, axis=1)
    sh = jnp.tanh(jnp.dot(sw1_ref[...], sinp, preferred_element_type=f32)
                  + sb1_ref[...])
    delta = (jnp.dot(sw2_ref[...], sh, preferred_element_type=f32)
             + sb2_ref[...])

    sig = jax.nn.sigmoid(nd)
    rows = []
    for b in range(bs):
        rows.append(ht_r[b] * sig[b] + delta[:, b * ALPHA:(b + 1) * ALPHA])
    out_ref[...] = jnp.stack(rows, axis=0).reshape(bs, 1, D, ALPHA)


def kernel(x, h, prev_messages, w_conn, decay_logit, primitives_state,
           hebbian_traces, state_w1, state_b1, state_w2, state_b2,
           msg_w1, msg_b1, msg_w2, msg_b2,
           mod_w1, mod_b1, mod_w2, mod_b2,
           neuron_id, conn_indices, border_indices):
    bs = x.shape[0]
    N = NC * C

    xv = x.transpose(0, 2, 3, 1)
    ht = h.transpose(0, 1, 3, 2)
    prevt = prev_messages.transpose(0, 1, 3, 2)
    wct = w_conn.transpose(0, 1, 3, 2)
    hebbt = hebbian_traces.transpose(0, 1, 3, 2)
    primt = primitives_state.transpose(0, 1, 3, 2)
    nidt = neuron_id.transpose(0, 2, 1)
    connt = conn_indices.transpose(0, 2, 1)
    m1t = mod_w1.transpose(2, 1, 0).reshape(MOD_IN * HMOD, N)
    m2t = mod_w2.transpose(2, 1, 0).reshape(MOD_OUT * HMOD, N)
    mb1t = mod_b1.transpose(1, 0)
    mb2t = mod_b2.transpose(1, 0)
    sb1c = state_b1.reshape(HS, 1)
    sb2c = state_b2.reshape(D, 1)
    mb1c = msg_b1.reshape(HM, 1)
    mb2c = msg_b2.reshape(D, 1)

    grid = (NC,)
    body = functools.partial(_body, bs=bs)
    out = pl.pallas_call(
        body,
        grid=grid,
        in_specs=[
            pl.BlockSpec((bs, ALPHA, D, NC), lambda i: (0, 0, 0, 0)),
            pl.BlockSpec((bs, 1, D, C), lambda i: (0, i, 0, 0)),
            pl.BlockSpec((1, K, C), lambda i: (i, 0, 0)),
            pl.BlockSpec((bs, 1, K, C), lambda i: (0, i, 0, 0)),
            pl.BlockSpec((bs, 1, D, C), lambda i: (0, i, 0, 0)),
            pl.BlockSpec((bs, 1, K, C), lambda i: (0, i, 0, 0)),
            pl.BlockSpec((bs, 1, 1, C), lambda i: (0, i, 0, 0)),
            pl.BlockSpec((bs, 1, D, C), lambda i: (0, i, 0, 0)),
            pl.BlockSpec((1, D, C), lambda i: (i, 0, 0)),
            pl.BlockSpec((MOD_IN * HMOD, CH), lambda i: (0, 2 * i + 1)),
            pl.BlockSpec((HMOD, CH), lambda i: (0, 2 * i + 1)),
            pl.BlockSpec((MOD_OUT * HMOD, CH), lambda i: (0, 2 * i + 1)),
            pl.BlockSpec((MOD_OUT, CH), lambda i: (0, 2 * i + 1)),
            pl.BlockSpec(state_w1.shape, lambda i: (0, 0)),
            pl.BlockSpec((HS, 1), lambda i: (0, 0)),
            pl.BlockSpec(state_w2.shape, lambda i: (0, 0)),
            pl.BlockSpec((D, 1), lambda i: (0, 0)),
            pl.BlockSpec(msg_w1.shape, lambda i: (0, 0)),
            pl.BlockSpec((HM, 1), lambda i: (0, 0)),
            pl.BlockSpec(msg_w2.shape, lambda i: (0, 0)),
            pl.BlockSpec((D, 1), lambda i: (0, 0)),
        ],
        out_specs=pl.BlockSpec((bs, 1, D, ALPHA), lambda i: (0, i, 0, 0)),
        out_shape=jax.ShapeDtypeStruct((bs, NC, D, ALPHA), jnp.float32),
    )(xv, ht, connt, wct, prevt, hebbt,
      decay_logit.reshape(bs, NC, 1, C), primt, nidt,
      m1t, mb1t, m2t, mb2t,
      state_w1, sb1c, state_w2, sb2c,
      msg_w1, mb1c, msg_w2, mb2c)
    return out.transpose(0, 1, 3, 2)
